# baseline (speedup 1.0000x reference)
import functools

import jax
import jax.numpy as jnp
from jax import lax
from jax.experimental import pallas as pl
from jax.experimental.pallas import tpu as pltpu
from jax.experimental.pallas import tpu_sc as plsc

D = 32
_INV_SQRT_D = 1.0 / (32.0 ** 0.5)


_T = 2560


def _attn_body(h_ref, wq_ref, wk_ref, a_ref):
    hb = h_ref[...]
    hT = hb.T
    qT = jnp.dot(wq_ref[...], hT, preferred_element_type=jnp.float32)
    kT = jnp.dot(wk_ref[...], hT,
                 preferred_element_type=jnp.float32) * _INV_SQRT_D
    e3 = jnp.exp(qT[:, None, :] * kT[None, :, :])
    denom = jnp.sum(e3, axis=0)
    gT = hT / denom
    aT = jnp.sum(e3 * gT[None, :, :], axis=1)
    a_ref[...] = aT.T


def _node_attention(h_pad, wq, wk):
    grid = (h_pad.shape[0] + _T - 1) // _T
    n_pad = grid * _T
    return pl.pallas_call(
        _attn_body,
        grid=(grid,),
        in_specs=[
            pl.BlockSpec((_T, D), lambda i: (i, 0)),
            pl.BlockSpec((D, D), lambda i: (0, 0)),
            pl.BlockSpec((D, D), lambda i: (0, 0)),
        ],
        out_specs=pl.BlockSpec((_T, D), lambda i: (i, 0)),
        out_shape=jax.ShapeDtypeStruct((n_pad, D), jnp.float32),
        compiler_params=pltpu.CompilerParams(
            dimension_semantics=("arbitrary",)),
    )(h_pad, wq, wk)


_CH = 128
_NW = 32
_NSUB = 16
_CW = 16


_NBUF = 4


def _edge_kernel(n_acc, n_chunks):
    rpt = n_acc // _NSUB
    epw = n_chunks * _CH
    n_grp = n_chunks // _NBUF
    mesh = plsc.VectorSubcoreMesh(core_axis_name="c", subcore_axis_name="s")

    @functools.partial(
        pl.kernel,
        out_type=[
            jax.ShapeDtypeStruct((2, n_acc, D), jnp.float32),
            jax.ShapeDtypeStruct((2, n_acc, _CW), jnp.float32),
        ],
        mesh=mesh,
        scratch_types=[
            pltpu.VMEM((epw,), jnp.int32),
            pltpu.VMEM((epw,), jnp.int32),
            [pltpu.VMEM((_CH, D), jnp.float32)] * _NBUF,
            pltpu.VMEM((_CH, _CW), jnp.float32),
            pltpu.VMEM_SHARED((n_acc, D), jnp.float32),
            pltpu.VMEM_SHARED((n_acc, _CW), jnp.float32),
            [pltpu.SemaphoreType.DMA] * _NBUF,
            [pltpu.SemaphoreType.DMA] * _NBUF,
            [pltpu.SemaphoreType.DMA] * _NBUF,
        ],
        compiler_params=pltpu.CompilerParams(use_tc_tiling_on_sc=False,
                                            skip_device_barrier=True),
    )
    def edge(a_hbm, ei_hbm, zsum_hbm, zcnt_hbm, ones_hbm,
             psum_hbm, pcnt_hbm,
             src_v, dst_v, rows, ones_v, ssum, scnt, gsem, ssem, csem):
        cid = lax.axis_index("c")
        sid = lax.axis_index("s")
        wid = cid * _NSUB + sid
        row0 = sid * rpt
        c1 = pltpu.async_copy(zsum_hbm.at[pl.ds(row0, rpt)],
                              ssum.at[pl.ds(row0, rpt)], ssem[0])
        c2 = pltpu.async_copy(zcnt_hbm.at[pl.ds(row0, rpt)],
                              scnt.at[pl.ds(row0, rpt)], csem[0])
        c3 = pltpu.async_copy(ones_hbm, ones_v, csem[1])
        c4 = pltpu.async_copy(ei_hbm.at[0, pl.ds(wid * epw, epw)], src_v,
                              gsem[0])
        c5 = pltpu.async_copy(ei_hbm.at[1, pl.ds(wid * epw, epw)], dst_v,
                              gsem[1])
        c1.wait()
        c2.wait()
        c3.wait()
        c4.wait()
        c5.wait()
        plsc.subcore_barrier()

        def _gather(c, b):
            pltpu.async_copy(a_hbm.at[src_v.at[pl.ds(c * _CH, _CH)]],
                             rows[b], gsem[b])

        def _scatter(c, b):
            dix = dst_v.at[pl.ds(c * _CH, _CH)]
            pltpu.async_copy(rows[b], ssum.at[dix], ssem[b], add=True)
            pltpu.async_copy(ones_v, scnt.at[dix], csem[b], add=True)

        def _wait(buf, sem):
            if buf is ones_v:
                pltpu.make_async_copy(ones_hbm, buf, sem).wait()
            else:
                pltpu.make_async_copy(a_hbm.at[src_v.at[pl.ds(0, _CH)]],
                                      buf, sem).wait()

        for b in range(_NBUF):
            _gather(b, b)

        def group(i, carry):
            c0 = i * _NBUF
            for b in range(_NBUF):
                _wait(rows[b], gsem[b])
                _scatter(c0 + b, b)
            for b in range(_NBUF):
                _wait(rows[b], ssem[b])
                _wait(ones_v, csem[b])

                @pl.when(i + 1 < n_grp)
                def _():
                    _gather(c0 + _NBUF + b, b)
            return carry

        lax.fori_loop(0, n_grp, group, 0)
        c0 = n_grp * _NBUF
        n_rem = n_chunks - c0
        for b in range(n_rem):
            _gather(c0 + b, b)
        for b in range(n_rem):
            _wait(rows[b], gsem[b])
            _scatter(c0 + b, b)
        for b in range(n_rem):
            _wait(rows[b], ssem[b])
            _wait(ones_v, csem[b])
        plsc.subcore_barrier()
        pltpu.sync_copy(ssum.at[pl.ds(row0, rpt)],
                        psum_hbm.at[cid, pl.ds(row0, rpt)])
        pltpu.sync_copy(scnt.at[pl.ds(row0, rpt)],
                        pcnt_hbm.at[cid, pl.ds(row0, rpt)])

    return edge


def _combine_kernel(n_acc, n_out):
    base = (n_out // _NW) // 8 * 8
    tail0 = _NW * base
    tail = n_out - tail0
    mesh = plsc.VectorSubcoreMesh(core_axis_name="c", subcore_axis_name="s")

    @functools.partial(
        pl.kernel,
        out_type=jax.ShapeDtypeStruct((n_out * D,), jnp.float32),
        mesh=mesh,
        scratch_types=[
            pltpu.VMEM((base + tail, D), jnp.float32),
            pltpu.VMEM((base + tail, D), jnp.float32),
            pltpu.VMEM((base + tail, _CW), jnp.float32),
            pltpu.VMEM((base + tail, _CW), jnp.float32),
            pltpu.VMEM(((base + tail) * D,), jnp.float32),
            pltpu.SemaphoreType.DMA,
            pltpu.SemaphoreType.DMA,
        ],
        compiler_params=pltpu.CompilerParams(use_tc_tiling_on_sc=False,
                                            skip_device_barrier=True),
    )
    def combine(psum_hbm, pcnt_hbm, out_hbm, p0, p1, c0, c1, o, psem, csem):
        cid = lax.axis_index("c")
        sid = lax.axis_index("s")
        wid = cid * _NSUB + sid
        nr = jnp.where(wid == 0, base + tail, base)
        row0 = jnp.where(wid == 0, 0, wid * base + tail)

        def do(nrows):
            cp0 = pltpu.async_copy(psum_hbm.at[0, pl.ds(row0, nrows)],
                                   p0.at[pl.ds(0, nrows)], psem)
            cp1 = pltpu.async_copy(psum_hbm.at[1, pl.ds(row0, nrows)],
                                   p1.at[pl.ds(0, nrows)], psem)
            cc0 = pltpu.async_copy(pcnt_hbm.at[0, pl.ds(row0, nrows)],
                                   c0.at[pl.ds(0, nrows)], csem)
            cc1 = pltpu.async_copy(pcnt_hbm.at[1, pl.ds(row0, nrows)],
                                   c1.at[pl.ds(0, nrows)], csem)
            cp0.wait()
            cp1.wait()
            cc0.wait()
            cc1.wait()

            def one_row(r):
                inv = 1.0 / jnp.maximum(c0[r, pl.ds(0, 16)]
                                        + c1[r, pl.ds(0, 16)], 1.0)
                o[pl.ds(r * D, 16)] = (p0[r, pl.ds(0, 16)]
                                       + p1[r, pl.ds(0, 16)]) * inv
                o[pl.ds(r * D + 16, 16)] = (p0[r, pl.ds(16, 16)]
                                            + p1[r, pl.ds(16, 16)]) * inv

            @plsc.parallel_loop(0, nrows, unroll=8)
            def _(r):
                one_row(r)
            pltpu.sync_copy(o.at[pl.ds(0, nrows * D)],
                            out_hbm.at[pl.ds(row0 * D, nrows * D)])

        @pl.when(wid == 0)
        def _():
            do(base + tail)

        @pl.when(wid != 0)
        def _():
            do(base)

    return combine


def kernel(h, edge_index, Wq, Wk):
    n, d = h.shape
    e = edge_index.shape[1]

    n_acc = ((n + 1 + 127) // 128) * 128
    epw = ((e + _NW * _CH - 1) // (_NW * _CH)) * _CH
    n_chunks = epw // _CH
    e_pad = _NW * epw
    ei = jnp.pad(edge_index, ((0, 0), (0, e_pad - e)), constant_values=n)

    zsum = jnp.zeros((n_acc, D), jnp.float32)
    zcnt = jnp.zeros((n_acc, _CW), jnp.float32)
    ones = jnp.ones((_CH, _CW), jnp.float32)

    a = _node_attention(h, Wq, Wk)
    psum, pcnt = _edge_kernel(n_acc, n_chunks)(a, ei, zsum, zcnt, ones)
    return _combine_kernel(n_acc, n)(psum, pcnt).reshape(n, D)


# speedup vs baseline: 1.0033x; 1.0033x over previous
import functools

import jax
import jax.numpy as jnp
from jax import lax
from jax.experimental import pallas as pl
from jax.experimental.pallas import tpu as pltpu
from jax.experimental.pallas import tpu_sc as plsc

D = 32
_INV_SQRT_D = 1.0 / (32.0 ** 0.5)


_T = 2048


def _attn_body(h_ref, wq_ref, wk_ref, a_ref):
    hb = h_ref[...]
    hT = hb.T
    qT = jnp.dot(wq_ref[...], hT, preferred_element_type=jnp.float32)
    kT = jnp.dot(wk_ref[...], hT,
                 preferred_element_type=jnp.float32) * _INV_SQRT_D
    e3 = jnp.exp(qT[:, None, :] * kT[None, :, :])
    denom = jnp.sum(e3, axis=0)
    gT = hT / denom
    aT = jnp.sum(e3 * gT[None, :, :], axis=1)
    a_ref[...] = aT.T


def _node_attention(h_pad, wq, wk):
    grid = (h_pad.shape[0] + _T - 1) // _T
    n_pad = grid * _T
    return pl.pallas_call(
        _attn_body,
        grid=(grid,),
        in_specs=[
            pl.BlockSpec((_T, D), lambda i: (i, 0)),
            pl.BlockSpec((D, D), lambda i: (0, 0)),
            pl.BlockSpec((D, D), lambda i: (0, 0)),
        ],
        out_specs=pl.BlockSpec((_T, D), lambda i: (i, 0)),
        out_shape=jax.ShapeDtypeStruct((n_pad, D), jnp.float32),
        compiler_params=pltpu.CompilerParams(
            dimension_semantics=("arbitrary",)),
    )(h_pad, wq, wk)


_CH = 128
_NW = 32
_NSUB = 16
_CW = 16


_NBUF = 4


def _edge_kernel(n_acc, n_chunks):
    rpt = n_acc // _NSUB
    epw = n_chunks * _CH
    n_grp = n_chunks // _NBUF
    mesh = plsc.VectorSubcoreMesh(core_axis_name="c", subcore_axis_name="s")

    @functools.partial(
        pl.kernel,
        out_type=[
            jax.ShapeDtypeStruct((2, n_acc, D), jnp.float32),
            jax.ShapeDtypeStruct((2, n_acc, _CW), jnp.float32),
        ],
        mesh=mesh,
        scratch_types=[
            pltpu.VMEM((epw,), jnp.int32),
            pltpu.VMEM((epw,), jnp.int32),
            [pltpu.VMEM((_CH, D), jnp.float32)] * _NBUF,
            pltpu.VMEM((_CH, _CW), jnp.float32),
            pltpu.VMEM_SHARED((n_acc, D), jnp.float32),
            pltpu.VMEM_SHARED((n_acc, _CW), jnp.float32),
            [pltpu.SemaphoreType.DMA] * _NBUF,
            [pltpu.SemaphoreType.DMA] * _NBUF,
            [pltpu.SemaphoreType.DMA] * _NBUF,
        ],
        compiler_params=pltpu.CompilerParams(use_tc_tiling_on_sc=False,
                                            skip_device_barrier=True),
    )
    def edge(a_hbm, ei_hbm, zsum_hbm, zcnt_hbm, ones_hbm,
             psum_hbm, pcnt_hbm,
             src_v, dst_v, rows, ones_v, ssum, scnt, gsem, ssem, csem):
        cid = lax.axis_index("c")
        sid = lax.axis_index("s")
        wid = cid * _NSUB + sid
        row0 = sid * rpt
        c1 = pltpu.async_copy(zsum_hbm.at[pl.ds(row0, rpt)],
                              ssum.at[pl.ds(row0, rpt)], ssem[0])
        c2 = pltpu.async_copy(zcnt_hbm.at[pl.ds(row0, rpt)],
                              scnt.at[pl.ds(row0, rpt)], csem[0])
        c3 = pltpu.async_copy(ones_hbm, ones_v, csem[1])
        c4 = pltpu.async_copy(ei_hbm.at[0, pl.ds(wid * epw, epw)], src_v,
                              gsem[0])
        c5 = pltpu.async_copy(ei_hbm.at[1, pl.ds(wid * epw, epw)], dst_v,
                              gsem[1])
        c1.wait()
        c2.wait()
        c3.wait()
        c4.wait()
        c5.wait()
        plsc.subcore_barrier()

        def _gather(c, b):
            pltpu.async_copy(a_hbm.at[src_v.at[pl.ds(c * _CH, _CH)]],
                             rows[b], gsem[b])

        def _scatter(c, b):
            dix = dst_v.at[pl.ds(c * _CH, _CH)]
            pltpu.async_copy(rows[b], ssum.at[dix], ssem[b], add=True)
            pltpu.async_copy(ones_v, scnt.at[dix], csem[b], add=True)

        def _wait(buf, sem):
            if buf is ones_v:
                pltpu.make_async_copy(ones_hbm, buf, sem).wait()
            else:
                pltpu.make_async_copy(a_hbm.at[src_v.at[pl.ds(0, _CH)]],
                                      buf, sem).wait()

        for b in range(_NBUF):
            _gather(b, b)

        def group(i, carry):
            c0 = i * _NBUF
            for b in range(_NBUF):
                _wait(rows[b], gsem[b])
                _scatter(c0 + b, b)
            for b in range(_NBUF):
                _wait(rows[b], ssem[b])
                _wait(ones_v, csem[b])

                @pl.when(i + 1 < n_grp)
                def _():
                    _gather(c0 + _NBUF + b, b)
            return carry

        lax.fori_loop(0, n_grp, group, 0)
        c0 = n_grp * _NBUF
        n_rem = n_chunks - c0
        for b in range(n_rem):
            _gather(c0 + b, b)
        for b in range(n_rem):
            _wait(rows[b], gsem[b])
            _scatter(c0 + b, b)
        for b in range(n_rem):
            _wait(rows[b], ssem[b])
            _wait(ones_v, csem[b])
        plsc.subcore_barrier()
        pltpu.sync_copy(ssum.at[pl.ds(row0, rpt)],
                        psum_hbm.at[cid, pl.ds(row0, rpt)])
        pltpu.sync_copy(scnt.at[pl.ds(row0, rpt)],
                        pcnt_hbm.at[cid, pl.ds(row0, rpt)])

    return edge


def _combine_kernel(n_acc, n_out):
    base = (n_out // _NW) // 8 * 8
    tail0 = _NW * base
    tail = n_out - tail0
    mesh = plsc.VectorSubcoreMesh(core_axis_name="c", subcore_axis_name="s")

    @functools.partial(
        pl.kernel,
        out_type=jax.ShapeDtypeStruct((n_out * D,), jnp.float32),
        mesh=mesh,
        scratch_types=[
            pltpu.VMEM((base + tail, D), jnp.float32),
            pltpu.VMEM((base + tail, D), jnp.float32),
            pltpu.VMEM((base + tail, _CW), jnp.float32),
            pltpu.VMEM((base + tail, _CW), jnp.float32),
            pltpu.VMEM(((base + tail) * D,), jnp.float32),
            pltpu.SemaphoreType.DMA,
            pltpu.SemaphoreType.DMA,
        ],
        compiler_params=pltpu.CompilerParams(use_tc_tiling_on_sc=False,
                                            skip_device_barrier=True),
    )
    def combine(psum_hbm, pcnt_hbm, out_hbm, p0, p1, c0, c1, o, psem, csem):
        cid = lax.axis_index("c")
        sid = lax.axis_index("s")
        wid = cid * _NSUB + sid
        nr = jnp.where(wid == 0, base + tail, base)
        row0 = jnp.where(wid == 0, 0, wid * base + tail)

        def do(nrows):
            cp0 = pltpu.async_copy(psum_hbm.at[0, pl.ds(row0, nrows)],
                                   p0.at[pl.ds(0, nrows)], psem)
            cp1 = pltpu.async_copy(psum_hbm.at[1, pl.ds(row0, nrows)],
                                   p1.at[pl.ds(0, nrows)], psem)
            cc0 = pltpu.async_copy(pcnt_hbm.at[0, pl.ds(row0, nrows)],
                                   c0.at[pl.ds(0, nrows)], csem)
            cc1 = pltpu.async_copy(pcnt_hbm.at[1, pl.ds(row0, nrows)],
                                   c1.at[pl.ds(0, nrows)], csem)
            cp0.wait()
            cp1.wait()
            cc0.wait()
            cc1.wait()

            def one_row(r):
                inv = 1.0 / jnp.maximum(c0[r, pl.ds(0, 16)]
                                        + c1[r, pl.ds(0, 16)], 1.0)
                o[pl.ds(r * D, 16)] = (p0[r, pl.ds(0, 16)]
                                       + p1[r, pl.ds(0, 16)]) * inv
                o[pl.ds(r * D + 16, 16)] = (p0[r, pl.ds(16, 16)]
                                            + p1[r, pl.ds(16, 16)]) * inv

            @plsc.parallel_loop(0, nrows, unroll=4)
            def _(r):
                one_row(r)
            pltpu.sync_copy(o.at[pl.ds(0, nrows * D)],
                            out_hbm.at[pl.ds(row0 * D, nrows * D)])

        @pl.when(wid == 0)
        def _():
            do(base + tail)

        @pl.when(wid != 0)
        def _():
            do(base)

    return combine


def kernel(h, edge_index, Wq, Wk):
    n, d = h.shape
    e = edge_index.shape[1]

    n_acc = ((n + 1 + 127) // 128) * 128
    epw = ((e + _NW * _CH - 1) // (_NW * _CH)) * _CH
    n_chunks = epw // _CH
    e_pad = _NW * epw
    ei = jnp.pad(edge_index, ((0, 0), (0, e_pad - e)), constant_values=n)

    zsum = jnp.zeros((n_acc, D), jnp.float32)
    zcnt = jnp.zeros((n_acc, _CW), jnp.float32)
    ones = jnp.ones((_CH, _CW), jnp.float32)

    a = _node_attention(h, Wq, Wk)
    psum, pcnt = _edge_kernel(n_acc, n_chunks)(a, ei, zsum, zcnt, ones)
    return _combine_kernel(n_acc, n)(psum, pcnt).reshape(n, D)
